# baseline (speedup 1.0000x reference)
import functools
import math

import jax
import jax.numpy as jnp
from jax import lax
from jax.experimental import pallas as pl
from jax.experimental.pallas import tpu as pltpu
from jax.experimental.pallas import tpu_sc as plsc

D_MODEL = 64
SCALE = math.sqrt(D_MODEL)

NC = 2
NS = 16
NW = NC * NS
C = 128
LANES = 16
SUB = 8
NBUF = 4


def _emb_body(xt_hbm, table_hbm, out_hbm, idx_v, rin, tbuf, gsems, ssems):
    wid = lax.axis_index("s") * NC + lax.axis_index("c")
    g_tot = xt_hbm.shape[1]
    n_steps = g_tot // NBUF
    jtiles = out_hbm.shape[2]

    pltpu.sync_copy(xt_hbm.at[wid], idx_v)

    iota = lax.iota(jnp.int32, LANES)
    toks = [iota + (jj * LANES) for jj in range(C // LANES)]

    def start_gather(b, g):
        pltpu.async_copy(table_hbm.at[idx_v.at[g]], rin.at[b], gsems.at[b])

    def wait_gather(b, g):
        pltpu.make_async_copy(table_hbm.at[idx_v.at[g]], rin.at[b],
                              gsems.at[b]).wait()

    def start_scatter(b, g):
        c = wid * g_tot + g
        h = c // jtiles
        j = lax.rem(c, jtiles)
        for i in range(D_MODEL // SUB):
            pltpu.async_copy(tbuf.at[b, pl.ds(i * SUB, SUB)],
                             out_hbm.at[h, i, j], ssems.at[b])

    def wait_scatter(b, g):
        c = wid * g_tot + g
        h = c // jtiles
        j = lax.rem(c, jtiles)
        for i in range(D_MODEL // SUB):
            pltpu.make_async_copy(tbuf.at[b, pl.ds(i * SUB, SUB)],
                                  out_hbm.at[h, i, j], ssems.at[b]).wait()

    def transpose_scale(b):
        src = rin.at[b]

        def per_d(d, carry):
            d_vec = jnp.full((LANES,), d, jnp.int32)
            for jj in range(C // LANES):
                vals = plsc.load_gather(src, [toks[jj], d_vec])
                tbuf[b, d, pl.ds(jj * LANES, LANES)] = vals * SCALE
            return carry

        lax.fori_loop(0, D_MODEL, per_d, 0, unroll=4)

    for b in range(NBUF):
        start_gather(b, b)

    for b in range(NBUF):
        wait_gather(b, b)
        transpose_scale(b)
        start_gather(b, b + NBUF)
        start_scatter(b, b)

    def step(s, carry):
        g0 = s * NBUF
        for b in range(NBUF):
            g = g0 + b
            wait_gather(b, g)
            wait_scatter(b, g - NBUF)
            transpose_scale(b)
            start_gather(b, g + NBUF)
            start_scatter(b, g)
        return carry

    lax.fori_loop(1, n_steps - 1, step, 0)

    g0 = (n_steps - 1) * NBUF
    for b in range(NBUF):
        g = g0 + b
        wait_gather(b, g)
        wait_scatter(b, g - NBUF)
        transpose_scale(b)
        start_scatter(b, g)
    for b in range(NBUF):
        wait_scatter(b, g0 + b)


def kernel(x, embedding):
    bsz, h = x.shape
    n = bsz * h
    assert n % (NW * C * NBUF) == 0
    g_per_w = n // (NW * C)
    jtiles = bsz // C
    xt = x.T.astype(jnp.int32).reshape(NW, g_per_w, C)

    mesh = plsc.VectorSubcoreMesh(core_axis_name="c", subcore_axis_name="s")
    out5 = pl.kernel(
        _emb_body,
        out_type=jax.ShapeDtypeStruct(
            (h, D_MODEL // SUB, jtiles, SUB, C), jnp.float32),
        mesh=mesh,
        scratch_types=[
            pltpu.VMEM((g_per_w, C), jnp.int32),
            pltpu.VMEM((NBUF, C, D_MODEL), jnp.float32),
            pltpu.VMEM((NBUF, D_MODEL, C), jnp.float32),
            pltpu.SemaphoreType.DMA((NBUF,)),
            pltpu.SemaphoreType.DMA((NBUF,)),
        ],
        compiler_params=pltpu.CompilerParams(
            use_tc_tiling_on_sc=False, needs_layout_passes=False),
    )(xt, embedding)
    return out5.transpose(2, 4, 0, 1, 3).reshape(bsz, h, D_MODEL)
